# baseline (speedup 1.0000x reference)
import functools

import jax
import jax.numpy as jnp
from jax import lax
from jax.experimental import pallas as pl
from jax.experimental.pallas import tpu as pltpu
from jax.experimental.pallas import tpu_sc as plsc

_NUM_CLASSES = 3
_IMG = 800.0
_SCORE_THRESH = 0.05
_NMS_THRESH = 0.5
_DETS = 100
_CLIP = 4.135166556742356

_N = 20000
_NPAD = 20480
_ROWS = _NPAD // 128
_TROWS = 2 * _ROWS

_NEG_INF = float("-inf")


def _nms_body(i, carry, x1a, y1a, x2a, y2a, areas, idx2d, cls_band, col_iota, row_iota):
    sw, acc = carry
    m = jnp.max(sw)
    picked = m > _NEG_INF
    eq = sw == m
    j = jnp.min(jnp.where(eq, idx2d, jnp.int32(2**30)))
    eqj = idx2d == j
    zero = jnp.zeros_like(x1a)
    bx1 = jnp.sum(jnp.where(eqj, x1a, zero))
    by1 = jnp.sum(jnp.where(eqj, y1a, zero))
    bx2 = jnp.sum(jnp.where(eqj, x2a, zero))
    by2 = jnp.sum(jnp.where(eqj, y2a, zero))
    barea = (bx2 - bx1) * (by2 - by1)
    ltx = jnp.maximum(bx1, x1a)
    lty = jnp.maximum(by1, y1a)
    rbx = jnp.minimum(bx2, x2a)
    rby = jnp.minimum(by2, y2a)
    iw = jnp.maximum(rbx - ltx, 0.0)
    ih = jnp.maximum(rby - lty, 0.0)
    inter = iw * ih
    iou = inter / (barea + areas - inter + 1e-9)
    same_cls = cls_band == (j < _ROWS * 128)
    suppress = ((iou > _NMS_THRESH) & same_cls) | eqj
    sw = jnp.where(suppress, _NEG_INF, sw)
    lab = jnp.where(j < _ROWS * 128, 1.0, 2.0)
    val = jnp.where(row_iota == 0, bx1,
          jnp.where(row_iota == 1, by1,
          jnp.where(row_iota == 2, bx2,
          jnp.where(row_iota == 3, by2,
          jnp.where(row_iota == 4, m, lab)))))
    val = jnp.where(picked, val, 0.0)
    acc = jnp.where(col_iota == i, val, acc)
    return sw, acc


def _fused_kernel(logits_ref, reg_ref, prop_ref, out_ref):
    l0 = logits_ref[0]
    l1 = logits_ref[1]
    l2 = logits_ref[2]
    m = jnp.maximum(jnp.maximum(l0, l1), l2)
    e0 = jnp.exp(l0 - m)
    e1 = jnp.exp(l1 - m)
    e2 = jnp.exp(l2 - m)
    denom = e0 + e1 + e2
    scores = [e1 / denom, e2 / denom]

    px1 = prop_ref[0]
    py1 = prop_ref[1]
    px2 = prop_ref[2]
    py2 = prop_ref[3]
    widths = px2 - px1
    heights = py2 - py1
    ctr_x = px1 + 0.5 * widths
    ctr_y = py1 + 0.5 * heights

    x1s, y1s, x2s, y2s, sws = [], [], [], [], []
    for ci, c in enumerate((1, 2)):
        dx = reg_ref[4 * c + 0] * 0.1
        dy = reg_ref[4 * c + 1] * 0.1
        dw = jnp.minimum(reg_ref[4 * c + 2] * 0.2, _CLIP)
        dh = jnp.minimum(reg_ref[4 * c + 3] * 0.2, _CLIP)
        pcx = dx * widths + ctr_x
        pcy = dy * heights + ctr_y
        pw = jnp.exp(dw) * widths
        ph = jnp.exp(dh) * heights
        x1 = jnp.clip(pcx - 0.5 * pw, 0.0, _IMG)
        y1 = jnp.clip(pcy - 0.5 * ph, 0.0, _IMG)
        x2 = jnp.clip(pcx + 0.5 * pw, 0.0, _IMG)
        y2 = jnp.clip(pcy + 0.5 * ph, 0.0, _IMG)
        s = scores[ci]
        valid = (s > _SCORE_THRESH) & ((x2 - x1) >= 0.01) & ((y2 - y1) >= 0.01)
        sws.append(jnp.where(valid, s, _NEG_INF))
        x1s.append(x1)
        y1s.append(y1)
        x2s.append(x2)
        y2s.append(y2)

    x1a = jnp.concatenate(x1s, axis=0)
    y1a = jnp.concatenate(y1s, axis=0)
    x2a = jnp.concatenate(x2s, axis=0)
    y2a = jnp.concatenate(y2s, axis=0)
    sw = jnp.concatenate(sws, axis=0)
    areas = (x2a - x1a) * (y2a - y1a)

    rid = lax.broadcasted_iota(jnp.int32, (_TROWS, 128), 0)
    cid = lax.broadcasted_iota(jnp.int32, (_TROWS, 128), 1)
    idx2d = rid * 128 + cid
    cls_band = rid < _ROWS

    row8 = lax.broadcasted_iota(jnp.int32, (8, 128), 0)
    col8 = lax.broadcasted_iota(jnp.int32, (8, 128), 1)
    acc0 = jnp.zeros((8, 128), jnp.float32)

    body = functools.partial(
        _nms_body, x1a=x1a, y1a=y1a, x2a=x2a, y2a=y2a, areas=areas,
        idx2d=idx2d, cls_band=cls_band, col_iota=col8, row_iota=row8)
    _, acc = lax.fori_loop(0, _DETS, body, (sw, acc0))
    out_ref[...] = acc


def _run(logits_t, reg_t, prop_t, interpret=False):
    return pl.pallas_call(
        _fused_kernel,
        out_shape=jax.ShapeDtypeStruct((8, 128), jnp.float32),
        interpret=interpret,
    )(logits_t, reg_t, prop_t)


def kernel(class_logits, box_regression, proposals):
    return kernel_sc(class_logits, box_regression, proposals)


def kernel_tc(class_logits, box_regression, proposals):
    pad = _NPAD - _N
    lt = jnp.pad(class_logits, ((0, pad), (0, 0))).T.reshape(_NUM_CLASSES, _ROWS, 128)
    rt = jnp.pad(box_regression, ((0, pad), (0, 0))).T.reshape(4 * _NUM_CLASSES, _ROWS, 128)
    pt = jnp.pad(proposals, ((0, pad), (0, 0))).T.reshape(4, _ROWS, 128)
    out = _run(lt, rt, pt)
    boxes = out[0:4, :_DETS].T
    nm_scores = out[4, :_DETS]
    labels = out[5, :_DETS].astype(jnp.int32)
    return boxes, nm_scores, labels


_CAND = 40960
_CHUNK = 2560
_TAIL = _N - 7 * _CHUNK
_L2N = 160
_L3N = 10
_RG0 = 0
_PP0 = 12 * _CHUNK


def _iota16():
    return lax.broadcasted_iota(jnp.int32, (16,), 0)


def _sc_kernel(lg_hbm, rg_hbm, pp_hbm, boxes_hbm, scores_hbm, labels_hbm,
               bufA, bufB, swb, browb, t1, t2, t3,
               accall, boxrow, obox, osc, olb, swp, browp):
    t = lax.axis_index("s")
    iota = _iota16()
    neg = jnp.full((16,), _NEG_INF, jnp.float32)
    zv = jnp.zeros((16,), jnp.float32)
    lane0 = iota == 0

    c = 1 + t // 8
    n0 = (t % 8) * _CHUNK
    tail = (t % 8) == 7

    @pl.when(tail)
    def _dma_tail():
        pltpu.sync_copy(rg_hbm.at[pl.ds(n0 * 12, _TAIL * 12)],
                        bufA.at[pl.ds(_RG0, _TAIL * 12)])
        pltpu.sync_copy(pp_hbm.at[pl.ds(n0 * 4, _TAIL * 4)],
                        bufA.at[pl.ds(_PP0, _TAIL * 4)])
        pltpu.sync_copy(lg_hbm.at[pl.ds(n0 * 3, _TAIL * 3)],
                        bufB.at[pl.ds(0, _TAIL * 3)])

    @pl.when(jnp.logical_not(tail))
    def _dma_full():
        pltpu.sync_copy(rg_hbm.at[pl.ds(n0 * 12, _CHUNK * 12)],
                        bufA.at[pl.ds(_RG0, _CHUNK * 12)])
        pltpu.sync_copy(pp_hbm.at[pl.ds(n0 * 4, _CHUNK * 4)],
                        bufA.at[pl.ds(_PP0, _CHUNK * 4)])
        pltpu.sync_copy(lg_hbm.at[pl.ds(n0 * 3, _CHUNK * 3)],
                        bufB.at[pl.ds(0, _CHUNK * 3)])

    rbase = 4 * c

    def decode_body(i, _):
        col = i * 16 + iota
        l0 = plsc.load_gather(bufB, [col * 3])
        l1 = plsc.load_gather(bufB, [col * 3 + 1])
        l2 = plsc.load_gather(bufB, [col * 3 + 2])
        r12 = col * 12 + rbase
        dx = plsc.load_gather(bufA, [r12]) * 0.1
        dy = plsc.load_gather(bufA, [r12 + 1]) * 0.1
        dw = jnp.minimum(plsc.load_gather(bufA, [r12 + 2]) * 0.2, _CLIP)
        dh = jnp.minimum(plsc.load_gather(bufA, [r12 + 3]) * 0.2, _CLIP)
        p4 = _PP0 + col * 4
        px1 = plsc.load_gather(bufA, [p4])
        py1 = plsc.load_gather(bufA, [p4 + 1])
        px2 = plsc.load_gather(bufA, [p4 + 2])
        py2 = plsc.load_gather(bufA, [p4 + 3])
        w = px2 - px1
        h = py2 - py1
        cx = px1 + 0.5 * w
        cy = py1 + 0.5 * h
        pcx = dx * w + cx
        pcy = dy * h + cy
        pw = jnp.exp(dw) * w
        ph = jnp.exp(dh) * h
        x1 = jnp.clip(pcx - 0.5 * pw, 0.0, _IMG)
        y1 = jnp.clip(pcy - 0.5 * ph, 0.0, _IMG)
        x2 = jnp.clip(pcx + 0.5 * pw, 0.0, _IMG)
        y2 = jnp.clip(pcy + 0.5 * ph, 0.0, _IMG)
        mx = jnp.maximum(jnp.maximum(l0, l1), l2)
        e0 = jnp.exp(l0 - mx)
        e1 = jnp.exp(l1 - mx)
        e2 = jnp.exp(l2 - mx)
        den = e0 + e1 + e2
        s = jnp.where(c == 1, e1, e2) / den
        valid = ((s > _SCORE_THRESH) & ((x2 - x1) >= 0.01)
                 & ((y2 - y1) >= 0.01) & (n0 + col < _N))
        sw = jnp.where(valid, s, neg)
        plsc.store_scatter(swb, [col], sw)
        r8 = col * 8
        plsc.store_scatter(browb, [r8 + 0], x1)
        plsc.store_scatter(browb, [r8 + 1], y1)
        plsc.store_scatter(browb, [r8 + 2], x2)
        plsc.store_scatter(browb, [r8 + 3], y2)
        return 0

    lax.fori_loop(0, _CHUNK // 16, decode_body, 0)

    base = t * _CHUNK
    pltpu.sync_copy(swb, swp.at[pl.ds(base, _CHUNK)])
    pltpu.sync_copy(browb, browp.at[pl.ds(base * 8, _CHUNK * 8)])
    plsc.subcore_barrier()

    @pl.when(t == 0)
    def _nms():
        pltpu.sync_copy(swp, bufA)

        for k in range(672 // 16):
            accall[pl.ds(k * 16, 16)] = zv
        for k in range(112 // 16):
            osc[pl.ds(k * 16, 16)] = zv
            olb[pl.ds(k * 16, 16)] = jnp.zeros((16,), jnp.int32)
        for k in range(400 // 16):
            fl = k * 16 + iota
            plsc.store_scatter(obox, [fl >> 2, fl & 3], zv)

        def l1_body(i, _):
            a = bufA[pl.ds(i * 16, 16)]
            for mi in range(1, 16):
                a = jnp.maximum(a, bufA[pl.ds(i * 16 + mi * _CHUNK, 16)])
            t1[pl.ds(i * 16, 16)] = a
            return 0

        lax.fori_loop(0, _CHUNK // 16, l1_body, 0)

        for i in range(_L2N // 16):
            a = t1[pl.ds(i * 16, 16)]
            for mi in range(1, 16):
                a = jnp.maximum(a, t1[pl.ds(i * 16 + mi * _L2N, 16)])
            t2[pl.ds(i * 16, 16)] = a

        t3v = neg
        for p in range(_L3N):
            t3v = jnp.where(iota == p, jnp.max(t2[pl.ds(p * 16, 16)]), t3v)
        t3[...] = t3v

        m0 = jnp.max(t3[...])

        def pop_cond(carry):
            m, nacc = carry
            return (m > _NEG_INF) & (nacc < _DETS)

        def pop_body(carry):
            m, nacc = carry
            t3c = t3[...]
            e3 = jnp.max(plsc.all_reduce_ffs(t3c == m))
            v2 = plsc.load_gather(t2, [e3 * 16 + iota])
            e2 = jnp.max(plsc.all_reduce_ffs(v2 == m))
            w = e3 * 16 + e2
            v1 = plsc.load_gather(t1, [w + _L2N * iota])
            e1 = jnp.max(plsc.all_reduce_ffs(v1 == m))
            v = w + _L2N * e1
            v0 = plsc.load_gather(bufA, [v + _CHUNK * iota])
            e0 = jnp.max(plsc.all_reduce_ffs(v0 == m))
            j = v + _CHUNK * e0
            pltpu.sync_copy(browp.at[pl.ds(j * 8, 16)], boxrow)
            z16 = iota * 0
            bx1 = plsc.load_gather(boxrow, [z16])
            by1 = plsc.load_gather(boxrow, [z16 + 1])
            bx2 = plsc.load_gather(boxrow, [z16 + 2])
            by2 = plsc.load_gather(boxrow, [z16 + 3])
            barea = (bx2 - bx1) * (by2 - by1)
            clsj = jnp.where(j < _NPAD, 1.0, 2.0)
            def iou_chunk(k, bad):
                a1v = accall[pl.ds(k * 16, 16)]
                b1v = accall[pl.ds(112 + k * 16, 16)]
                a2v = accall[pl.ds(224 + k * 16, 16)]
                b2v = accall[pl.ds(336 + k * 16, 16)]
                aav = accall[pl.ds(448 + k * 16, 16)]
                aclv = accall[pl.ds(560 + k * 16, 16)]
                ltx = jnp.maximum(a1v, bx1)
                lty = jnp.maximum(b1v, by1)
                rbx = jnp.minimum(a2v, bx2)
                rby = jnp.minimum(b2v, by2)
                iw = jnp.maximum(rbx - ltx, 0.0)
                ih = jnp.maximum(rby - lty, 0.0)
                inter = iw * ih
                iou = inter / (aav + barea - inter + 1e-9)
                b = (iou > _NMS_THRESH) & (aclv == clsj)
                return bad | b

            nchunks = (nacc + 15) // 16
            bad = lax.fori_loop(0, nchunks, iou_chunk,
                                jnp.zeros((16,), jnp.bool_))
            accept = jnp.logical_not(jnp.any(bad))

            @pl.when(accept)
            def _store():
                vals = jnp.where(iota == 0, bx1,
                       jnp.where(iota == 1, by1,
                       jnp.where(iota == 2, bx2,
                       jnp.where(iota == 3, by2,
                       jnp.where(iota == 4, barea, clsj)))))
                plsc.store_scatter(accall, [nacc + 112 * iota], vals,
                                   mask=iota < 6)
                plsc.store_scatter(obox, [jnp.broadcast_to(nacc, (16,)), iota],
                                   vals, mask=iota < 4)
                plsc.store_scatter(osc, [jnp.broadcast_to(nacc, (16,))],
                                   jnp.broadcast_to(m, (16,)), mask=lane0)
                plsc.store_scatter(olb, [jnp.broadcast_to(nacc, (16,))],
                                   jnp.broadcast_to(clsj.astype(jnp.int32),
                                                    (16,)), mask=lane0)

            v0n = jnp.where(iota == e0, neg, v0)
            l1v = jnp.max(v0n)
            v1n = jnp.where(iota == e1, l1v, v1)
            l2v = jnp.max(v1n)
            v2n = jnp.where(iota == e2, l2v, v2)
            l3v = jnp.max(v2n)
            t3n = jnp.where(iota == e3, l3v, t3c)
            t3[...] = t3n
            plsc.store_scatter(bufA, [jnp.broadcast_to(j, (16,))], neg,
                               mask=lane0)
            plsc.store_scatter(t1, [jnp.broadcast_to(v, (16,))],
                               jnp.broadcast_to(l1v, (16,)), mask=lane0)
            plsc.store_scatter(t2, [jnp.broadcast_to(w, (16,))],
                               jnp.broadcast_to(l2v, (16,)), mask=lane0)
            m2 = jnp.max(t3n)
            return m2, nacc + jnp.where(accept, 1, 0)

        lax.while_loop(pop_cond, pop_body, (m0, jnp.int32(0)))
        pltpu.sync_copy(obox, boxes_hbm)
        pltpu.sync_copy(osc.at[pl.ds(0, _DETS)], scores_hbm)
        pltpu.sync_copy(olb.at[pl.ds(0, _DETS)], labels_hbm)


def _make_sc_call():
    mesh = plsc.VectorSubcoreMesh(core_axis_name="c", subcore_axis_name="s",
                                  num_cores=1)
    f32 = jnp.float32
    return pl.kernel(
        _sc_kernel,
        out_type=(
            jax.ShapeDtypeStruct((_DETS, 4), f32),
            jax.ShapeDtypeStruct((_DETS,), f32),
            jax.ShapeDtypeStruct((_DETS,), jnp.int32),
        ),
        mesh=mesh,
        compiler_params=pltpu.CompilerParams(needs_layout_passes=False),
        scratch_types=[
            pltpu.VMEM((16 * _CHUNK,), f32),
            pltpu.VMEM((3 * _CHUNK,), f32),
            pltpu.VMEM((_CHUNK,), f32),
            pltpu.VMEM((_CHUNK * 8,), f32),
            pltpu.VMEM((_CHUNK,), f32),
            pltpu.VMEM((_L2N,), f32),
            pltpu.VMEM((16,), f32),
            pltpu.VMEM((672,), f32),
            pltpu.VMEM((16,), f32),
            pltpu.VMEM((_DETS, 4), f32),
            pltpu.VMEM((112,), f32),
            pltpu.VMEM((112,), jnp.int32),
            pltpu.VMEM_SHARED((_CAND,), f32),
            pltpu.VMEM_SHARED((_CAND * 8 + 8,), f32),
        ],
    )


def kernel_sc(class_logits, box_regression, proposals):
    boxes, nm_scores, labels = _make_sc_call()(
        class_logits.reshape(-1), box_regression.reshape(-1),
        proposals.reshape(-1))
    return boxes, nm_scores, labels


# speedup vs baseline: 1.0030x; 1.0030x over previous
import functools

import jax
import jax.numpy as jnp
from jax import lax
from jax.experimental import pallas as pl
from jax.experimental.pallas import tpu as pltpu
from jax.experimental.pallas import tpu_sc as plsc

_NUM_CLASSES = 3
_IMG = 800.0
_SCORE_THRESH = 0.05
_NMS_THRESH = 0.5
_DETS = 100
_CLIP = 4.135166556742356

_N = 20000
_NPAD = 20480
_ROWS = _NPAD // 128
_TROWS = 2 * _ROWS

_NEG_INF = float("-inf")


def _nms_body(i, carry, x1a, y1a, x2a, y2a, areas, idx2d, cls_band, col_iota, row_iota):
    sw, acc = carry
    m = jnp.max(sw)
    picked = m > _NEG_INF
    eq = sw == m
    j = jnp.min(jnp.where(eq, idx2d, jnp.int32(2**30)))
    eqj = idx2d == j
    zero = jnp.zeros_like(x1a)
    bx1 = jnp.sum(jnp.where(eqj, x1a, zero))
    by1 = jnp.sum(jnp.where(eqj, y1a, zero))
    bx2 = jnp.sum(jnp.where(eqj, x2a, zero))
    by2 = jnp.sum(jnp.where(eqj, y2a, zero))
    barea = (bx2 - bx1) * (by2 - by1)
    ltx = jnp.maximum(bx1, x1a)
    lty = jnp.maximum(by1, y1a)
    rbx = jnp.minimum(bx2, x2a)
    rby = jnp.minimum(by2, y2a)
    iw = jnp.maximum(rbx - ltx, 0.0)
    ih = jnp.maximum(rby - lty, 0.0)
    inter = iw * ih
    iou = inter / (barea + areas - inter + 1e-9)
    same_cls = cls_band == (j < _ROWS * 128)
    suppress = ((iou > _NMS_THRESH) & same_cls) | eqj
    sw = jnp.where(suppress, _NEG_INF, sw)
    lab = jnp.where(j < _ROWS * 128, 1.0, 2.0)
    val = jnp.where(row_iota == 0, bx1,
          jnp.where(row_iota == 1, by1,
          jnp.where(row_iota == 2, bx2,
          jnp.where(row_iota == 3, by2,
          jnp.where(row_iota == 4, m, lab)))))
    val = jnp.where(picked, val, 0.0)
    acc = jnp.where(col_iota == i, val, acc)
    return sw, acc


def _fused_kernel(logits_ref, reg_ref, prop_ref, out_ref):
    l0 = logits_ref[0]
    l1 = logits_ref[1]
    l2 = logits_ref[2]
    m = jnp.maximum(jnp.maximum(l0, l1), l2)
    e0 = jnp.exp(l0 - m)
    e1 = jnp.exp(l1 - m)
    e2 = jnp.exp(l2 - m)
    denom = e0 + e1 + e2
    scores = [e1 / denom, e2 / denom]

    px1 = prop_ref[0]
    py1 = prop_ref[1]
    px2 = prop_ref[2]
    py2 = prop_ref[3]
    widths = px2 - px1
    heights = py2 - py1
    ctr_x = px1 + 0.5 * widths
    ctr_y = py1 + 0.5 * heights

    x1s, y1s, x2s, y2s, sws = [], [], [], [], []
    for ci, c in enumerate((1, 2)):
        dx = reg_ref[4 * c + 0] * 0.1
        dy = reg_ref[4 * c + 1] * 0.1
        dw = jnp.minimum(reg_ref[4 * c + 2] * 0.2, _CLIP)
        dh = jnp.minimum(reg_ref[4 * c + 3] * 0.2, _CLIP)
        pcx = dx * widths + ctr_x
        pcy = dy * heights + ctr_y
        pw = jnp.exp(dw) * widths
        ph = jnp.exp(dh) * heights
        x1 = jnp.clip(pcx - 0.5 * pw, 0.0, _IMG)
        y1 = jnp.clip(pcy - 0.5 * ph, 0.0, _IMG)
        x2 = jnp.clip(pcx + 0.5 * pw, 0.0, _IMG)
        y2 = jnp.clip(pcy + 0.5 * ph, 0.0, _IMG)
        s = scores[ci]
        valid = (s > _SCORE_THRESH) & ((x2 - x1) >= 0.01) & ((y2 - y1) >= 0.01)
        sws.append(jnp.where(valid, s, _NEG_INF))
        x1s.append(x1)
        y1s.append(y1)
        x2s.append(x2)
        y2s.append(y2)

    x1a = jnp.concatenate(x1s, axis=0)
    y1a = jnp.concatenate(y1s, axis=0)
    x2a = jnp.concatenate(x2s, axis=0)
    y2a = jnp.concatenate(y2s, axis=0)
    sw = jnp.concatenate(sws, axis=0)
    areas = (x2a - x1a) * (y2a - y1a)

    rid = lax.broadcasted_iota(jnp.int32, (_TROWS, 128), 0)
    cid = lax.broadcasted_iota(jnp.int32, (_TROWS, 128), 1)
    idx2d = rid * 128 + cid
    cls_band = rid < _ROWS

    row8 = lax.broadcasted_iota(jnp.int32, (8, 128), 0)
    col8 = lax.broadcasted_iota(jnp.int32, (8, 128), 1)
    acc0 = jnp.zeros((8, 128), jnp.float32)

    body = functools.partial(
        _nms_body, x1a=x1a, y1a=y1a, x2a=x2a, y2a=y2a, areas=areas,
        idx2d=idx2d, cls_band=cls_band, col_iota=col8, row_iota=row8)
    _, acc = lax.fori_loop(0, _DETS, body, (sw, acc0))
    out_ref[...] = acc


def _run(logits_t, reg_t, prop_t, interpret=False):
    return pl.pallas_call(
        _fused_kernel,
        out_shape=jax.ShapeDtypeStruct((8, 128), jnp.float32),
        interpret=interpret,
    )(logits_t, reg_t, prop_t)


def kernel(class_logits, box_regression, proposals):
    return kernel_sc(class_logits, box_regression, proposals)


def kernel_tc(class_logits, box_regression, proposals):
    pad = _NPAD - _N
    lt = jnp.pad(class_logits, ((0, pad), (0, 0))).T.reshape(_NUM_CLASSES, _ROWS, 128)
    rt = jnp.pad(box_regression, ((0, pad), (0, 0))).T.reshape(4 * _NUM_CLASSES, _ROWS, 128)
    pt = jnp.pad(proposals, ((0, pad), (0, 0))).T.reshape(4, _ROWS, 128)
    out = _run(lt, rt, pt)
    boxes = out[0:4, :_DETS].T
    nm_scores = out[4, :_DETS]
    labels = out[5, :_DETS].astype(jnp.int32)
    return boxes, nm_scores, labels


_CAND = 40960
_CHUNK = 2560
_TAIL = _N - 7 * _CHUNK
_L2N = 160
_L3N = 10
_RG0 = 0
_PP0 = 12 * _CHUNK


def _iota16():
    return lax.broadcasted_iota(jnp.int32, (16,), 0)


def _sc_kernel(lg_hbm, rg_hbm, pp_hbm, boxes_hbm, scores_hbm, labels_hbm,
               bufA, bufB, swb, browb, t1, t2, t3,
               accall, boxrow, obox, osc, olb, swp, browp):
    t = lax.axis_index("s")
    iota = _iota16()
    neg = jnp.full((16,), _NEG_INF, jnp.float32)
    zv = jnp.zeros((16,), jnp.float32)
    lane0 = iota == 0

    c = 1 + t // 8
    n0 = (t % 8) * _CHUNK
    tail = (t % 8) == 7

    @pl.when(tail)
    def _dma_tail():
        pltpu.sync_copy(rg_hbm.at[pl.ds(n0 * 12, _TAIL * 12)],
                        bufA.at[pl.ds(_RG0, _TAIL * 12)])
        pltpu.sync_copy(pp_hbm.at[pl.ds(n0 * 4, _TAIL * 4)],
                        bufA.at[pl.ds(_PP0, _TAIL * 4)])
        pltpu.sync_copy(lg_hbm.at[pl.ds(n0 * 3, _TAIL * 3)],
                        bufB.at[pl.ds(0, _TAIL * 3)])

    @pl.when(jnp.logical_not(tail))
    def _dma_full():
        pltpu.sync_copy(rg_hbm.at[pl.ds(n0 * 12, _CHUNK * 12)],
                        bufA.at[pl.ds(_RG0, _CHUNK * 12)])
        pltpu.sync_copy(pp_hbm.at[pl.ds(n0 * 4, _CHUNK * 4)],
                        bufA.at[pl.ds(_PP0, _CHUNK * 4)])
        pltpu.sync_copy(lg_hbm.at[pl.ds(n0 * 3, _CHUNK * 3)],
                        bufB.at[pl.ds(0, _CHUNK * 3)])

    rbase = 4 * c

    def decode_body(i, _):
        col = i * 16 + iota
        lc = plsc.load_gather(bufB, [col * 3 + c])
        l0 = plsc.load_gather(bufB, [col * 3])
        lb = plsc.load_gather(bufB, [col * 3 + (3 - c)])
        r12 = col * 12 + rbase
        dx = plsc.load_gather(bufA, [r12]) * 0.1
        dy = plsc.load_gather(bufA, [r12 + 1]) * 0.1
        dw = jnp.minimum(plsc.load_gather(bufA, [r12 + 2]) * 0.2, _CLIP)
        dh = jnp.minimum(plsc.load_gather(bufA, [r12 + 3]) * 0.2, _CLIP)
        p4 = _PP0 + col * 4
        px1 = plsc.load_gather(bufA, [p4])
        py1 = plsc.load_gather(bufA, [p4 + 1])
        px2 = plsc.load_gather(bufA, [p4 + 2])
        py2 = plsc.load_gather(bufA, [p4 + 3])
        w = px2 - px1
        h = py2 - py1
        cx = px1 + 0.5 * w
        cy = py1 + 0.5 * h
        pcx = dx * w + cx
        pcy = dy * h + cy
        pw = jnp.exp(dw) * w
        ph = jnp.exp(dh) * h
        x1 = jnp.clip(pcx - 0.5 * pw, 0.0, _IMG)
        y1 = jnp.clip(pcy - 0.5 * ph, 0.0, _IMG)
        x2 = jnp.clip(pcx + 0.5 * pw, 0.0, _IMG)
        y2 = jnp.clip(pcy + 0.5 * ph, 0.0, _IMG)
        s = 1.0 / (1.0 + jnp.exp(l0 - lc) + jnp.exp(lb - lc))
        valid = ((s > _SCORE_THRESH) & ((x2 - x1) >= 0.01)
                 & ((y2 - y1) >= 0.01) & (n0 + col < _N))
        sw = jnp.where(valid, s, neg)
        plsc.store_scatter(swb, [col], sw)
        r8 = col * 8
        plsc.store_scatter(browb, [r8 + 0], x1)
        plsc.store_scatter(browb, [r8 + 1], y1)
        plsc.store_scatter(browb, [r8 + 2], x2)
        plsc.store_scatter(browb, [r8 + 3], y2)
        return 0

    lax.fori_loop(0, _CHUNK // 16, decode_body, 0)

    base = t * _CHUNK
    pltpu.sync_copy(swb, swp.at[pl.ds(base, _CHUNK)])
    pltpu.sync_copy(browb, browp.at[pl.ds(base * 8, _CHUNK * 8)])
    plsc.subcore_barrier()

    @pl.when(t == 0)
    def _nms():
        pltpu.sync_copy(swp, bufA)

        for k in range(672 // 16):
            accall[pl.ds(k * 16, 16)] = zv
        for k in range(112 // 16):
            osc[pl.ds(k * 16, 16)] = zv
            olb[pl.ds(k * 16, 16)] = jnp.zeros((16,), jnp.int32)
        for k in range(400 // 16):
            fl = k * 16 + iota
            plsc.store_scatter(obox, [fl >> 2, fl & 3], zv)

        def l1_body(i, _):
            a = bufA[pl.ds(i * 16, 16)]
            for mi in range(1, 16):
                a = jnp.maximum(a, bufA[pl.ds(i * 16 + mi * _CHUNK, 16)])
            t1[pl.ds(i * 16, 16)] = a
            return 0

        lax.fori_loop(0, _CHUNK // 16, l1_body, 0)

        for i in range(_L2N // 16):
            a = t1[pl.ds(i * 16, 16)]
            for mi in range(1, 16):
                a = jnp.maximum(a, t1[pl.ds(i * 16 + mi * _L2N, 16)])
            t2[pl.ds(i * 16, 16)] = a

        t3v = neg
        for p in range(_L3N):
            t3v = jnp.where(iota == p, jnp.max(t2[pl.ds(p * 16, 16)]), t3v)
        t3[...] = t3v

        m0 = jnp.max(t3[...])

        def pop_cond(carry):
            m, nacc = carry
            return (m > _NEG_INF) & (nacc < _DETS)

        def pop_body(carry):
            m, nacc = carry
            t3c = t3[...]
            e3 = jnp.max(plsc.all_reduce_ffs(t3c == m))
            v2 = plsc.load_gather(t2, [e3 * 16 + iota])
            e2 = jnp.max(plsc.all_reduce_ffs(v2 == m))
            w = e3 * 16 + e2
            v1 = plsc.load_gather(t1, [w + _L2N * iota])
            e1 = jnp.max(plsc.all_reduce_ffs(v1 == m))
            v = w + _L2N * e1
            v0 = plsc.load_gather(bufA, [v + _CHUNK * iota])
            e0 = jnp.max(plsc.all_reduce_ffs(v0 == m))
            j = v + _CHUNK * e0
            pltpu.sync_copy(browp.at[pl.ds(j * 8, 16)], boxrow)
            z16 = iota * 0
            bx1 = plsc.load_gather(boxrow, [z16])
            by1 = plsc.load_gather(boxrow, [z16 + 1])
            bx2 = plsc.load_gather(boxrow, [z16 + 2])
            by2 = plsc.load_gather(boxrow, [z16 + 3])
            barea = (bx2 - bx1) * (by2 - by1)
            clsj = jnp.where(j < _NPAD, 1.0, 2.0)
            def iou_chunk(k, bad):
                a1v = accall[pl.ds(k * 16, 16)]
                b1v = accall[pl.ds(112 + k * 16, 16)]
                a2v = accall[pl.ds(224 + k * 16, 16)]
                b2v = accall[pl.ds(336 + k * 16, 16)]
                aav = accall[pl.ds(448 + k * 16, 16)]
                aclv = accall[pl.ds(560 + k * 16, 16)]
                ltx = jnp.maximum(a1v, bx1)
                lty = jnp.maximum(b1v, by1)
                rbx = jnp.minimum(a2v, bx2)
                rby = jnp.minimum(b2v, by2)
                iw = jnp.maximum(rbx - ltx, 0.0)
                ih = jnp.maximum(rby - lty, 0.0)
                inter = iw * ih
                iou = inter / (aav + barea - inter + 1e-9)
                b = (iou > _NMS_THRESH) & (aclv == clsj)
                return bad | b

            nchunks = (nacc + 15) // 16
            bad = lax.fori_loop(0, nchunks, iou_chunk,
                                jnp.zeros((16,), jnp.bool_))
            accept = jnp.logical_not(jnp.any(bad))

            @pl.when(accept)
            def _store():
                vals = jnp.where(iota == 0, bx1,
                       jnp.where(iota == 1, by1,
                       jnp.where(iota == 2, bx2,
                       jnp.where(iota == 3, by2,
                       jnp.where(iota == 4, barea, clsj)))))
                plsc.store_scatter(accall, [nacc + 112 * iota], vals,
                                   mask=iota < 6)
                plsc.store_scatter(obox, [jnp.broadcast_to(nacc, (16,)), iota],
                                   vals, mask=iota < 4)
                plsc.store_scatter(osc, [jnp.broadcast_to(nacc, (16,))],
                                   jnp.broadcast_to(m, (16,)), mask=lane0)
                plsc.store_scatter(olb, [jnp.broadcast_to(nacc, (16,))],
                                   jnp.broadcast_to(clsj.astype(jnp.int32),
                                                    (16,)), mask=lane0)

            v0n = jnp.where(iota == e0, neg, v0)
            l1v = jnp.max(v0n)
            v1n = jnp.where(iota == e1, l1v, v1)
            l2v = jnp.max(v1n)
            v2n = jnp.where(iota == e2, l2v, v2)
            l3v = jnp.max(v2n)
            t3n = jnp.where(iota == e3, l3v, t3c)
            t3[...] = t3n
            plsc.store_scatter(bufA, [jnp.broadcast_to(j, (16,))], neg,
                               mask=lane0)
            plsc.store_scatter(t1, [jnp.broadcast_to(v, (16,))],
                               jnp.broadcast_to(l1v, (16,)), mask=lane0)
            plsc.store_scatter(t2, [jnp.broadcast_to(w, (16,))],
                               jnp.broadcast_to(l2v, (16,)), mask=lane0)
            m2 = jnp.max(t3n)
            return m2, nacc + jnp.where(accept, 1, 0)

        lax.while_loop(pop_cond, pop_body, (m0, jnp.int32(0)))
        pltpu.sync_copy(obox, boxes_hbm)
        pltpu.sync_copy(osc.at[pl.ds(0, _DETS)], scores_hbm)
        pltpu.sync_copy(olb.at[pl.ds(0, _DETS)], labels_hbm)


def _make_sc_call():
    mesh = plsc.VectorSubcoreMesh(core_axis_name="c", subcore_axis_name="s",
                                  num_cores=1)
    f32 = jnp.float32
    return pl.kernel(
        _sc_kernel,
        out_type=(
            jax.ShapeDtypeStruct((_DETS, 4), f32),
            jax.ShapeDtypeStruct((_DETS,), f32),
            jax.ShapeDtypeStruct((_DETS,), jnp.int32),
        ),
        mesh=mesh,
        compiler_params=pltpu.CompilerParams(needs_layout_passes=False),
        scratch_types=[
            pltpu.VMEM((16 * _CHUNK,), f32),
            pltpu.VMEM((3 * _CHUNK,), f32),
            pltpu.VMEM((_CHUNK,), f32),
            pltpu.VMEM((_CHUNK * 8,), f32),
            pltpu.VMEM((_CHUNK,), f32),
            pltpu.VMEM((_L2N,), f32),
            pltpu.VMEM((16,), f32),
            pltpu.VMEM((672,), f32),
            pltpu.VMEM((16,), f32),
            pltpu.VMEM((_DETS, 4), f32),
            pltpu.VMEM((112,), f32),
            pltpu.VMEM((112,), jnp.int32),
            pltpu.VMEM_SHARED((_CAND,), f32),
            pltpu.VMEM_SHARED((_CAND * 8 + 8,), f32),
        ],
    )


def kernel_sc(class_logits, box_regression, proposals):
    boxes, nm_scores, labels = _make_sc_call()(
        class_logits.reshape(-1), box_regression.reshape(-1),
        proposals.reshape(-1))
    return boxes, nm_scores, labels
